# baseline (speedup 1.0000x reference)
import jax
import jax.numpy as jnp
from jax.experimental import pallas as pl
from jax.experimental.pallas import tpu as pltpu

HEADS = 2
HID = 32
G4 = 4 * HID
T = 12
T_OUT = 12
N = 512
NUM = 8192


def _leaky(x):
    return jnp.maximum(x, 0.2 * x)


def _fused_kernel(x24_ref, x24t_ref, at_ref, psrcT_ref, psrc_ref, pdst_ref,
                  wcwi_ref, pw0_ref, pw1_ref, b0a_ref, wh0_ref, wi1_ref,
                  wh1_ref, b1_ref, wfc_ref, bfc_ref, out_ref):
    xb = x24_ref[...]
    xb0 = xb[0:N, :]
    at = at_ref[...]
    asr = jnp.dot(psrcT_ref[...], x24t_ref[...],
                  preferred_element_type=jnp.float32)
    asc = jnp.dot(xb0, psrc_ref[...], preferred_element_type=jnp.float32)
    adc = jnp.dot(xb0, pdst_ref[...], preferred_element_type=jnp.float32)

    wcwi = wcwi_ref[...]
    pw0 = pw0_ref[...]
    pw1 = pw1_ref[...]
    ones_col = jnp.ones((N, 1), jnp.float32)
    b0a = b0a_ref[...]
    wh0 = wh0_ref[...]
    wi1 = wi1_ref[...]
    wh1 = wh1_ref[...]
    b1 = b1_ref[...]
    z = jnp.zeros((NUM, HID), jnp.float32)
    h1, c1, h2, c2 = z, z, z, z
    for t in range(T):
        x2 = xb0[:, 2 * t:2 * t + 2]
        acc = None
        for h, pw in ((0, pw0), (1, pw1)):
            c = 2 * t + h
            ad = adc[:, c:c + 1]
            dv = _leaky(ad + asc[:, c:c + 1])
            Em = jnp.exp(_leaky(ad + asr[c:c + 1, :]) - dv) * at
            hp = jax.lax.concatenate(
                [jnp.dot(x2, pw, preferred_element_type=jnp.float32),
                 ones_col], 1)
            r = jnp.dot(Em, hp, preferred_element_type=jnp.float32) + hp
            contrib = r[:, 0:G4] / (r[:, G4:G4 + 1] + 1e-16)
            acc = contrib if acc is None else acc + contrib
        xp_attn = 0.5 * acc
        xp_lin = jnp.dot(xb[N:, 2 * t:2 * t + 2], wcwi,
                         preferred_element_type=jnp.float32)
        xp = jax.lax.concatenate([xp_attn, xp_lin], 0)
        g = (xp + jnp.dot(h1, wh0, preferred_element_type=jnp.float32)
             + b0a)
        ii = jax.nn.sigmoid(g[:, 0:HID])
        ff = jax.nn.sigmoid(g[:, HID:2 * HID])
        gg = jnp.tanh(g[:, 2 * HID:3 * HID])
        oo = jax.nn.sigmoid(g[:, 3 * HID:4 * HID])
        c1 = ff * c1 + ii * gg
        h1 = oo * jnp.tanh(c1)
        g = (jnp.dot(h1, wi1, preferred_element_type=jnp.float32)
             + jnp.dot(h2, wh1, preferred_element_type=jnp.float32) + b1)
        ii = jax.nn.sigmoid(g[:, 0:HID])
        ff = jax.nn.sigmoid(g[:, HID:2 * HID])
        gg = jnp.tanh(g[:, 2 * HID:3 * HID])
        oo = jax.nn.sigmoid(g[:, 3 * HID:4 * HID])
        c2 = ff * c2 + ii * gg
        h2 = oo * jnp.tanh(c2)
    out_ref[...] = (jnp.dot(h2, wfc_ref[...],
                            preferred_element_type=jnp.float32) + bfc_ref[...])


def kernel(A_hat, X, W_gat, att_src, att_dst, b_gat, W_ih0, W_hh0, b_ih0,
           b_hh0, W_ih1, W_hh1, b_ih1, b_hh1, W_fc, b_fc):
    B, n, t, F = X.shape
    num = B * n
    x24 = X.reshape(num, t * F)
    x24t = x24[:n].T
    atT = (A_hat.T != 0).astype(jnp.float32)

    p_src = jnp.stack(
        [W_gat[:, h * HID:(h + 1) * HID] @ att_src[0, h] for h in range(HEADS)],
        axis=1)
    p_dst = jnp.stack(
        [W_gat[:, h * HID:(h + 1) * HID] @ att_dst[0, h] for h in range(HEADS)],
        axis=1)
    eyeT = jnp.eye(t, dtype=jnp.float32)
    Psrc = jnp.kron(eyeT, p_src)
    Pdst = jnp.kron(eyeT, p_dst)

    wi0 = W_ih0.T
    pw0 = W_gat[:, :HID] @ wi0
    pw1 = W_gat[:, HID:] @ wi0
    wcwi = (0.5 * (W_gat[:, :HID] + W_gat[:, HID:])) @ wi0
    b0a = (b_ih0 + b_hh0 + b_gat @ wi0)[None, :]
    wh0 = W_hh0.T
    wi1 = W_ih1.T
    wh1 = W_hh1.T
    b1 = (b_ih1 + b_hh1)[None, :]
    wfc = W_fc.T
    bfc = b_fc[None, :]

    out24 = pl.pallas_call(
        _fused_kernel,
        out_shape=jax.ShapeDtypeStruct((num, T_OUT * F), jnp.float32),
    )(x24, x24t, atT, Psrc.T, Psrc, Pdst, wcwi, pw0, pw1, b0a,
      wh0, wi1, wh1, b1, wfc, bfc)
    return out24.reshape(B, n, T_OUT, F)


# speedup vs baseline: 1.0656x; 1.0656x over previous
import jax
import jax.numpy as jnp
from jax.experimental import pallas as pl
from jax.experimental.pallas import tpu as pltpu

HEADS = 2
HID = 32
T = 12
T_OUT = 12
N = 512
NUM = 8192


def _leaky(x):
    return jnp.maximum(x, 0.2 * x)


def _fused_kernel(x24_ref, x24t_ref, at_ref, wgat_ref, psrcT_ref, psrc_ref,
                  pdst_ref, wcomb_ref, bg_ref, wi0_ref, wh0_ref, b0_ref,
                  wi1_ref, wh1_ref, b1_ref, wfc_ref, bfc_ref, out_ref):
    xb = x24_ref[...]
    xb0 = xb[0:N, :]
    at = at_ref[...]
    wg = wgat_ref[...]
    bg = bg_ref[...]
    wc = wcomb_ref[...]
    ones_col = jnp.ones((N, 1), jnp.float32)
    asr = jnp.dot(psrcT_ref[...], x24t_ref[...],
                  preferred_element_type=jnp.float32)
    asc = jnp.dot(xb0, psrc_ref[...], preferred_element_type=jnp.float32)
    adc = jnp.dot(xb0, pdst_ref[...], preferred_element_type=jnp.float32)

    wi0 = wi0_ref[...]
    wh0 = wh0_ref[...]
    b0 = b0_ref[...]
    wi1 = wi1_ref[...]
    wh1 = wh1_ref[...]
    b1 = b1_ref[...]
    z = jnp.zeros((NUM, HID), jnp.float32)
    h1, c1, h2, c2 = z, z, z, z
    for t in range(T):
        ht = jnp.dot(xb0[:, 2 * t:2 * t + 2], wg,
                     preferred_element_type=jnp.float32)
        acc = None
        for h in range(HEADS):
            c = 2 * t + h
            ad = adc[:, c:c + 1]
            dv = _leaky(ad + asc[:, c:c + 1])
            Em = jnp.exp(_leaky(ad + asr[c:c + 1, :]) - dv) * at
            hha = jax.lax.concatenate(
                [ht[:, HID * h:HID * h + HID], ones_col], 1)
            r = jnp.dot(Em, hha, preferred_element_type=jnp.float32) + hha
            agg = r[:, 0:HID] / (r[:, HID:HID + 1] + 1e-16)
            acc = agg if acc is None else acc + agg
        xattn = 0.5 * acc + bg
        xlin = (jnp.dot(xb[N:, 2 * t:2 * t + 2], wc,
                        preferred_element_type=jnp.float32) + bg)
        xt = jax.lax.concatenate([xattn, xlin], 0)
        g = (jnp.dot(xt, wi0, preferred_element_type=jnp.float32)
             + jnp.dot(h1, wh0, preferred_element_type=jnp.float32) + b0)
        ii = jax.nn.sigmoid(g[:, 0:HID])
        ff = jax.nn.sigmoid(g[:, HID:2 * HID])
        gg = jnp.tanh(g[:, 2 * HID:3 * HID])
        oo = jax.nn.sigmoid(g[:, 3 * HID:4 * HID])
        c1 = ff * c1 + ii * gg
        h1 = oo * jnp.tanh(c1)
        g = (jnp.dot(h1, wi1, preferred_element_type=jnp.float32)
             + jnp.dot(h2, wh1, preferred_element_type=jnp.float32) + b1)
        ii = jax.nn.sigmoid(g[:, 0:HID])
        ff = jax.nn.sigmoid(g[:, HID:2 * HID])
        gg = jnp.tanh(g[:, 2 * HID:3 * HID])
        oo = jax.nn.sigmoid(g[:, 3 * HID:4 * HID])
        c2 = ff * c2 + ii * gg
        h2 = oo * jnp.tanh(c2)
    out_ref[...] = (jnp.dot(h2, wfc_ref[...],
                            preferred_element_type=jnp.float32) + bfc_ref[...])


def kernel(A_hat, X, W_gat, att_src, att_dst, b_gat, W_ih0, W_hh0, b_ih0,
           b_hh0, W_ih1, W_hh1, b_ih1, b_hh1, W_fc, b_fc):
    B, n, t, F = X.shape
    num = B * n
    x24 = X.reshape(num, t * F)
    x24t = x24[:n].T
    atT = (A_hat.T != 0).astype(jnp.float32)

    p_src = jnp.stack(
        [W_gat[:, h * HID:(h + 1) * HID] @ att_src[0, h] for h in range(HEADS)],
        axis=1)
    p_dst = jnp.stack(
        [W_gat[:, h * HID:(h + 1) * HID] @ att_dst[0, h] for h in range(HEADS)],
        axis=1)
    eyeT = jnp.eye(t, dtype=jnp.float32)
    Psrc = jnp.kron(eyeT, p_src)
    Pdst = jnp.kron(eyeT, p_dst)
    Wcomb = 0.5 * (W_gat[:, :HID] + W_gat[:, HID:])
    bg = b_gat[None, :]

    wi0 = W_ih0.T
    wh0 = W_hh0.T
    b0 = (b_ih0 + b_hh0)[None, :]
    wi1 = W_ih1.T
    wh1 = W_hh1.T
    b1 = (b_ih1 + b_hh1)[None, :]
    wfc = W_fc.T
    bfc = b_fc[None, :]

    out24 = pl.pallas_call(
        _fused_kernel,
        out_shape=jax.ShapeDtypeStruct((num, T_OUT * F), jnp.float32),
    )(x24, x24t, atT, W_gat, Psrc.T, Psrc, Pdst, Wcomb, bg,
      wi0, wh0, b0, wi1, wh1, b1, wfc, bfc)
    return out24.reshape(B, n, T_OUT, F)
